# baseline (speedup 1.0000x reference)
import jax
import jax.numpy as jnp
from jax.experimental import pallas as pl

_H = 2


def _qpe_body(Th, map_t, actor_t, light_t, map_pe, actor_pe, light_pe,
              tpe_p, pos_m, pos_n, pos_l, pos_tp,
              map_o, actor_o, light_o):
    N = actor_t.shape[2]
    L = light_t.shape[2]
    D = map_t.shape[-1]

    map_o[...] = map_t[...] + (map_pe[...] + pos_m[...])[None]

    time_comb = (tpe_p[0, :Th] + pos_tp[0, :Th]).reshape(1, Th, 1, D)
    actor_comb = (actor_pe[:N] + pos_n[:N]).reshape(1, 1, N, D)
    actor_o[...] = actor_t[...] + actor_comb + time_comb

    light_comb = (light_pe[:L] + pos_l[:L]).reshape(1, 1, L, D)
    light_o[...] = light_t[...] + light_comb + time_comb


def _pack_halves(x, Th):
    Thp = (Th + 7) // 8 * 8
    pad = jnp.zeros((Thp - Th, x.shape[-1]), x.dtype)
    return jnp.stack([jnp.concatenate([x[:Th], pad]),
                      jnp.concatenate([x[Th:2 * Th], pad])])


def kernel(map_token, actor_token, light_token, map_pe_w, actor_pe_w,
           light_pe_w, time_pe_w, pos_enc):
    B, S, D = map_token.shape
    _, T, N, _ = actor_token.shape
    L = light_token.shape[2]
    Sh, Th = S // _H, T // _H
    Thp = (Th + 7) // 8 * 8
    Np = (N + 7) // 8 * 8
    Lp = (L + 7) // 8 * 8

    tpe_p = _pack_halves(time_pe_w, Th)
    pos_tp = _pack_halves(pos_enc, Th)

    import functools
    outs = pl.pallas_call(
        functools.partial(_qpe_body, Th),
        grid=(_H, B),
        in_specs=[
            pl.BlockSpec((1, Sh, D), lambda h, b: (b, h, 0)),
            pl.BlockSpec((1, Th, N, D), lambda h, b: (b, h, 0, 0)),
            pl.BlockSpec((1, Th, L, D), lambda h, b: (b, h, 0, 0)),
            pl.BlockSpec((Sh, D), lambda h, b: (h, 0)),
            pl.BlockSpec((Np, D), lambda h, b: (0, 0)),
            pl.BlockSpec((Lp, D), lambda h, b: (0, 0)),
            pl.BlockSpec((1, Thp, D), lambda h, b: (h, 0, 0)),
            pl.BlockSpec((Sh, D), lambda h, b: (h, 0)),
            pl.BlockSpec((Np, D), lambda h, b: (0, 0)),
            pl.BlockSpec((Lp, D), lambda h, b: (0, 0)),
            pl.BlockSpec((1, Thp, D), lambda h, b: (h, 0, 0)),
        ],
        out_specs=[
            pl.BlockSpec((1, Sh, D), lambda h, b: (b, h, 0)),
            pl.BlockSpec((1, Th, N, D), lambda h, b: (b, h, 0, 0)),
            pl.BlockSpec((1, Th, L, D), lambda h, b: (b, h, 0, 0)),
        ],
        out_shape=[
            jax.ShapeDtypeStruct((B, S, D), map_token.dtype),
            jax.ShapeDtypeStruct((B, T, N, D), actor_token.dtype),
            jax.ShapeDtypeStruct((B, T, L, D), light_token.dtype),
        ],
    )(map_token, actor_token, light_token, map_pe_w, actor_pe_w,
      light_pe_w, tpe_p, pos_enc, pos_enc, pos_enc, pos_tp)
    return tuple(outs)


# speedup vs baseline: 1.1217x; 1.1217x over previous
import jax
import jax.numpy as jnp
from jax.experimental import pallas as pl


def _qpe_body(map_t, actor_t, light_t, map_pe, actor_pe, light_pe, time_pe,
              pos, map_o, actor_o, light_o):
    S = map_t.shape[1]
    T = actor_t.shape[1]
    N = actor_t.shape[2]
    L = light_t.shape[2]
    D = map_t.shape[-1]

    pos_all = pos[...]
    map_o[...] = map_t[...] + (map_pe[...] + pos_all[:S])[None]

    time_comb = (time_pe[:T] + pos_all[:T]).reshape(1, T, 1, D)
    actor_comb = (actor_pe[:N] + pos_all[:N]).reshape(1, 1, N, D)
    actor_o[...] = actor_t[...] + actor_comb + time_comb

    light_comb = (light_pe[:L] + pos_all[:L]).reshape(1, 1, L, D)
    light_o[...] = light_t[...] + light_comb + time_comb


def kernel(map_token, actor_token, light_token, map_pe_w, actor_pe_w,
           light_pe_w, time_pe_w, pos_enc):
    B, S, D = map_token.shape
    _, T, N, _ = actor_token.shape
    L = light_token.shape[2]

    whole = lambda shape: pl.BlockSpec(shape, lambda b: (0,) * len(shape))
    outs = pl.pallas_call(
        _qpe_body,
        grid=(B,),
        in_specs=[
            pl.BlockSpec((1, S, D), lambda b: (b, 0, 0)),
            pl.BlockSpec((1, T, N, D), lambda b: (b, 0, 0, 0)),
            pl.BlockSpec((1, T, L, D), lambda b: (b, 0, 0, 0)),
            whole(map_pe_w.shape),
            whole(actor_pe_w.shape),
            whole(light_pe_w.shape),
            whole(time_pe_w.shape),
            whole(pos_enc.shape),
        ],
        out_specs=[
            pl.BlockSpec((1, S, D), lambda b: (b, 0, 0)),
            pl.BlockSpec((1, T, N, D), lambda b: (b, 0, 0, 0)),
            pl.BlockSpec((1, T, L, D), lambda b: (b, 0, 0, 0)),
        ],
        out_shape=[
            jax.ShapeDtypeStruct((B, S, D), map_token.dtype),
            jax.ShapeDtypeStruct((B, T, N, D), actor_token.dtype),
            jax.ShapeDtypeStruct((B, T, L, D), light_token.dtype),
        ],
    )(map_token, actor_token, light_token, map_pe_w, actor_pe_w,
      light_pe_w, time_pe_w, pos_enc)
    return tuple(outs)


# speedup vs baseline: 1.1442x; 1.0200x over previous
import jax
import jax.numpy as jnp
from jax.experimental import pallas as pl


def _qpe_body(map_t, actor_t, light_t, map_pe, actor_pe, light_pe, time_pe,
              pos, map_o, actor_o, light_o):
    S = map_t.shape[1]
    T = actor_t.shape[1]
    N = actor_t.shape[2]
    L = light_t.shape[2]
    D = map_t.shape[-1]

    pos_all = pos[...]
    map_o[...] = map_t[...] + (map_pe[...] + pos_all[:S])[None]

    time_comb = (time_pe[:T] + pos_all[:T]).reshape(1, T, 1, D)
    actor_comb = (actor_pe[:N] + pos_all[:N]).reshape(1, 1, N, D)
    actor_o[...] = actor_t[...] + actor_comb + time_comb

    light_comb = (light_pe[:L] + pos_all[:L]).reshape(1, 1, L, D)
    light_o[...] = light_t[...] + light_comb + time_comb


def kernel(map_token, actor_token, light_token, map_pe_w, actor_pe_w,
           light_pe_w, time_pe_w, pos_enc):
    B, S, D = map_token.shape
    _, T, N, _ = actor_token.shape
    L = light_token.shape[2]

    whole = lambda shape: pl.BlockSpec(shape, lambda b: (0,) * len(shape))
    G = 2
    outs = pl.pallas_call(
        _qpe_body,
        grid=(B // G,),
        in_specs=[
            pl.BlockSpec((G, S, D), lambda b: (b, 0, 0)),
            pl.BlockSpec((G, T, N, D), lambda b: (b, 0, 0, 0)),
            pl.BlockSpec((G, T, L, D), lambda b: (b, 0, 0, 0)),
            whole(map_pe_w.shape),
            whole(actor_pe_w.shape),
            whole(light_pe_w.shape),
            whole(time_pe_w.shape),
            whole(pos_enc.shape),
        ],
        out_specs=[
            pl.BlockSpec((G, S, D), lambda b: (b, 0, 0)),
            pl.BlockSpec((G, T, N, D), lambda b: (b, 0, 0, 0)),
            pl.BlockSpec((G, T, L, D), lambda b: (b, 0, 0, 0)),
        ],
        out_shape=[
            jax.ShapeDtypeStruct((B, S, D), map_token.dtype),
            jax.ShapeDtypeStruct((B, T, N, D), actor_token.dtype),
            jax.ShapeDtypeStruct((B, T, L, D), light_token.dtype),
        ],
    )(map_token, actor_token, light_token, map_pe_w, actor_pe_w,
      light_pe_w, time_pe_w, pos_enc)
    return tuple(outs)
